# baseline (speedup 1.0000x reference)
import functools
import math

import jax
import jax.numpy as jnp
from jax import lax
from jax.experimental import pallas as pl
from jax.experimental.pallas import tpu as pltpu
from jax.experimental.pallas import tpu_sc as plsc

D_MODEL = 64
VOCAB = 1000000
T_DIM = 200
B_DIM = 4096
SCALE = math.sqrt(D_MODEL)
NC, NS, L = 2, 16, 16
D_PER_CORE = D_MODEL // NC
B_PER_SUB = B_DIM // NS
TG = 8
NG = T_DIM // TG

_mesh = plsc.VectorSubcoreMesh(
    core_axis_name="c", subcore_axis_name="s", num_cores=NC, num_subcores=NS
)


@functools.partial(
    pl.kernel,
    mesh=_mesh,
    out_type=jax.ShapeDtypeStruct((T_DIM, D_MODEL, B_DIM), jnp.float32),
    scratch_types=[
        pltpu.VMEM((2, T_DIM, 128), jnp.int32),
        pltpu.VMEM((TG, B_PER_SUB), jnp.float32),
        pltpu.VMEM((TG, B_PER_SUB), jnp.float32),
        pltpu.VMEM_SHARED((VOCAB,), jnp.float32),
        pltpu.SemaphoreType.DMA,
        pltpu.SemaphoreType.DMA,
        pltpu.SemaphoreType.DMA,
        pltpu.SemaphoreType.DMA,
    ],
)
def _emb_kernel(
    xt_hbm, lut_hbm, out_hbm, idx_res, valsA, valsB, row_sh, gsem0, gsem1, ssem0, ssem1
):
    c = lax.axis_index("c")
    s = lax.axis_index("s")
    b0 = s * B_PER_SUB

    for h in range(2):
        pltpu.sync_copy(xt_hbm.at[:, pl.ds(b0 + h * 128, 128)], idx_res.at[h])

    def out_slice(g, d_global):
        return out_hbm.at[pl.ds(g * TG, TG), d_global, pl.ds(b0, B_PER_SUB)]

    def fire_gathers(g, buf, gsem):
        t0 = g * TG
        for tt in range(TG):
            for h in range(2):
                pltpu.async_copy(
                    row_sh.at[idx_res.at[h, t0 + tt]],
                    buf.at[tt, pl.ds(h * 128, 128)],
                    gsem,
                )

    def drain(hbm_side, vmem_buf, sem):
        pltpu.make_async_copy(hbm_side, vmem_buf, sem).wait()

    def scale(buf):
        for tt in range(TG):
            for q in range(B_PER_SUB // L):
                sl = pl.ds(q * L, L)
                buf[tt, sl] = buf[tt, sl] * SCALE

    def d_body(d, _):
        d_global = c * D_PER_CORE + d
        plsc.subcore_barrier()

        @pl.when(s == 0)
        def _stage_row():
            pltpu.sync_copy(lut_hbm.at[d_global], row_sh)

        plsc.subcore_barrier()

        fire_gathers(0, valsA, gsem0)

        def stage(g, buf, nbuf, gsem_b, gsem_n, ssem_b, ssem_n):
            @pl.when(g + 1 < NG)
            def _prefetch():
                @pl.when(g >= 1)
                def _wait_prev_store():
                    drain(out_slice(g - 1, d_global), nbuf, ssem_n)

                fire_gathers(g + 1, nbuf, gsem_n)

            drain(out_slice(g, d_global), buf, gsem_b)
            scale(buf)
            pltpu.async_copy(buf, out_slice(g, d_global), ssem_b)

        def g_body(g, _):
            stage(2 * g, valsA, valsB, gsem0, gsem1, ssem0, ssem1)
            stage(2 * g + 1, valsB, valsA, gsem1, gsem0, ssem1, ssem0)
            return _

        lax.fori_loop(0, NG // 2, g_body, 0)
        if NG % 2:
            stage(NG - 1, valsA, valsB, gsem0, gsem1, ssem0, ssem1)
        drain(out_slice(NG - 2, d_global), valsB, ssem1)
        drain(out_slice(NG - 1, d_global), valsA, ssem0)
        return _

    lax.fori_loop(0, D_PER_CORE, d_body, 0)


def kernel(x, lut):
    xt = x.astype(jnp.int32).T
    lut_t = lut.T
    out_t = _emb_kernel(xt, lut_t)
    return out_t.transpose(2, 0, 1)
